# Initial kernel scaffold; baseline (speedup 1.0000x reference)
import jax
import jax.numpy as jnp
from jax.experimental import pallas as pl


def kernel(x, edge_index, etype, weight, loop_w1, loop_w2):
    raise NotImplementedError("write your pallas kernel here")


# speedup vs baseline: 6.3093x; 6.3093x over previous
import functools

import jax
import jax.numpy as jnp
from jax import lax
from jax.experimental import pallas as pl
from jax.experimental.pallas import tpu as pltpu
from jax.experimental.pallas import tpu_sc as plsc

N = 10000
E = 320000
D = 128
R = 100

NC = 2
NS = 16
NW = NC * NS
CH = 25
EW = E // NW
RW = EW // CH
NPAD = 10112
ROWS_PER_TILE = NPAD // NS
WBCH = [(k * 40, 40) for k in range(15)] + [(600, 32)]
CCH = 100
RWC = EW // CCH

BM = 400
GRID = N // BM

_mesh = plsc.VectorSubcoreMesh(
    core_axis_name="c", subcore_axis_name="s", num_cores=NC, num_subcores=NS)


def _zero_buf(ref, nrows, width):
    def body(i, carry):
        for k in range(width // 16):
            ref[i, pl.ds(k * 16, 16)] = jnp.zeros((16,), jnp.float32)
        return carry
    lax.fori_loop(0, nrows, body, 0)


def _fill_ones(ref, nrows, width):
    def body(i, carry):
        for k in range(width // 16):
            ref[i, pl.ds(k * 16, 16)] = jnp.ones((16,), jnp.float32)
        return carry
    lax.fori_loop(0, nrows, body, 0)


@functools.partial(
    pl.kernel,
    out_type=(
        jax.ShapeDtypeStruct((NPAD, D), jnp.float32),
        jax.ShapeDtypeStruct((NPAD, D), jnp.float32),
        jax.ShapeDtypeStruct((NPAD, D), jnp.float32),
        jax.ShapeDtypeStruct((NPAD, D), jnp.float32),
    ),
    mesh=_mesh,
    scratch_types=[
        pltpu.VMEM((RWC, CCH), jnp.int32),
        pltpu.VMEM((CCH, D), jnp.float32),
        pltpu.VMEM((40, D), jnp.float32),
        pltpu.VMEM_SHARED((NPAD, D), jnp.float32),
        pltpu.SemaphoreType.DMA,
        pltpu.SemaphoreType.DMA,
    ],
)
def _sc_counts(src_hbm, dst_hbm, o0, i0, o1, i1,
               idxv, ones_v, zb, cnt_sp, sem0, sem1):
    c = lax.axis_index("c")
    s = lax.axis_index("s")
    w = s * NC + c

    _fill_ones(ones_v, CCH, D)
    _zero_buf(zb, 40, D)

    for phase in range(2):
        for off, ln in WBCH:
            pltpu.sync_copy(zb.at[pl.ds(0, ln)],
                            cnt_sp.at[pl.ds(s * ROWS_PER_TILE + off, ln)])
        pltpu.sync_copy((src_hbm if phase == 0 else dst_hbm).at[w], idxv)
        plsc.subcore_barrier()

        def body(t, carry):
            d0 = pltpu.async_copy(ones_v, cnt_sp.at[idxv.at[2 * t]],
                                  sem0, add=True)
            d1 = pltpu.async_copy(ones_v, cnt_sp.at[idxv.at[2 * t + 1]],
                                  sem1, add=True)
            d0.wait()
            d1.wait()
            return carry
        lax.fori_loop(0, RWC // 2, body, 0)
        plsc.subcore_barrier()

        outs = (o0, o1) if phase == 0 else (i0, i1)

        @pl.when(c == 0)
        def _():
            for off, ln in WBCH:
                base = s * ROWS_PER_TILE + off
                pltpu.sync_copy(cnt_sp.at[pl.ds(base, ln)], zb.at[pl.ds(0, ln)])
                pltpu.sync_copy(zb.at[pl.ds(0, ln)], outs[0].at[pl.ds(base, ln)])

        @pl.when(c == 1)
        def _():
            for off, ln in WBCH:
                base = s * ROWS_PER_TILE + off
                pltpu.sync_copy(cnt_sp.at[pl.ds(base, ln)], zb.at[pl.ds(0, ln)])
                pltpu.sync_copy(zb.at[pl.ds(0, ln)], outs[1].at[pl.ds(base, ln)])
        plsc.subcore_barrier()
        _zero_buf(zb, 40, D)


GRP = 40
NGRP = RW // GRP


@functools.partial(
    pl.kernel,
    out_type=(
        jax.ShapeDtypeStruct((NPAD, D), jnp.float32),
        jax.ShapeDtypeStruct((NPAD, D), jnp.float32),
    ),
    mesh=_mesh,
    scratch_types=[
        pltpu.VMEM((GRP, 2 * CH), jnp.int32),
        pltpu.VMEM((GRP, CH), jnp.int32),
        pltpu.VMEM((CH, D), jnp.float32),
        pltpu.VMEM((CH, D), jnp.float32),
        pltpu.VMEM((CH, D), jnp.float32),
        pltpu.VMEM((CH, D), jnp.float32),
        pltpu.VMEM((CH, D), jnp.float32),
        pltpu.VMEM((CH, D), jnp.float32),
        pltpu.VMEM((40, D), jnp.float32),
        pltpu.VMEM_SHARED((NPAD, D), jnp.float32),
        pltpu.VMEM_SHARED((R, D), jnp.float32),
        pltpu.SemaphoreType.DMA,
        pltpu.SemaphoreType.DMA,
        pltpu.SemaphoreType.DMA,
        pltpu.SemaphoreType.DMA,
        pltpu.SemaphoreType.DMA,
        pltpu.SemaphoreType.DMA,
        pltpu.SemaphoreType.DMA,
        pltpu.SemaphoreType.DMA,
        pltpu.SemaphoreType.DMA,
        pltpu.SemaphoreType.DMA,
    ],
)
def _sc_mp(xn_hbm, se_hbm, dst_hbm, w_hbm, accA, accB,
           seidx, didx, r0, r1, w0, w1, w2, w3, zb, acc_sp, wtab_sp,
           sgr0, sgr1, sgw0, sgw1, sgw2, sgw3, ssc0, ssc1, ssc2, ssc3):
    c = lax.axis_index("c")
    s = lax.axis_index("s")
    w = s * NC + c
    Rb = (r0, r1)
    Wb = (w0, w1, w2, w3)
    sgR = (sgr0, sgr1)
    sgW = (sgw0, sgw1, sgw2, sgw3)
    ssc = (ssc0, ssc1, ssc2, ssc3)

    _zero_buf(zb, 40, D)
    for off, ln in WBCH:
        pltpu.sync_copy(zb.at[pl.ds(0, ln)],
                        acc_sp.at[pl.ds(s * ROWS_PER_TILE + off, ln)])

    @pl.when(s == 0)
    def _():
        for k in range(R // 8):
            pltpu.sync_copy(w_hbm.at[pl.ds(k * 8, 8)], zb.at[pl.ds(0, 8)])
            pltpu.sync_copy(zb.at[pl.ds(0, 8)], wtab_sp.at[pl.ds(k * 8, 8)])
        pltpu.sync_copy(w_hbm.at[pl.ds(R - R % 8, R % 8)], zb.at[pl.ds(0, R % 8)])
        pltpu.sync_copy(zb.at[pl.ds(0, R % 8)],
                        wtab_sp.at[pl.ds(R - R % 8, R % 8)])
    plsc.subcore_barrier()

    def _mul(rbuf, wbuf):
        def mul(e, cc):
            for k in range(D // 16):
                sl = pl.ds(k * 16, 16)
                wbuf[e, sl] = rbuf[e, sl] * wbuf[e, sl]
            return cc
        lax.fori_loop(0, CH, mul, 0)

    def group(g, carry):
        pltpu.sync_copy(se_hbm.at[w, pl.ds(g * GRP, GRP)], seidx)
        pltpu.sync_copy(dst_hbm.at[w, pl.ds(g * GRP, GRP)], didx)
        for u in (0, 1):
            pltpu.async_copy(xn_hbm.at[seidx.at[u, pl.ds(0, CH)]], Rb[u], sgR[u])
            pltpu.async_copy(wtab_sp.at[seidx.at[u, pl.ds(CH, CH)]], Wb[u], sgW[u])

        def quad(t, carry2):
            for u in range(4):
                cc = 4 * t + u
                Rr, Wq = Rb[u % 2], Wb[u]
                pltpu.make_async_copy(
                    xn_hbm.at[seidx.at[cc, pl.ds(0, CH)]], Rr, sgR[u % 2]).wait()
                pltpu.make_async_copy(
                    wtab_sp.at[seidx.at[cc, pl.ds(CH, CH)]], Wq, sgW[u]).wait()
                _mul(Rr, Wq)

                @pl.when(cc + 2 < GRP)
                def _():
                    pltpu.async_copy(
                        xn_hbm.at[seidx.at[cc + 2, pl.ds(0, CH)]], Rr, sgR[u % 2])
                pltpu.async_copy(Wq, acc_sp.at[didx.at[cc]], ssc[u], add=True)
                q2 = (u + 2) % 4

                @pl.when(cc + 2 < GRP)
                def _():
                    @pl.when(cc >= 2)
                    def _():
                        pltpu.make_async_copy(
                            Wb[q2], acc_sp.at[didx.at[cc - 2]], ssc[q2]).wait()
                    pltpu.async_copy(
                        wtab_sp.at[seidx.at[cc + 2, pl.ds(CH, CH)]], Wb[q2], sgW[q2])
            return carry2
        lax.fori_loop(0, GRP // 4, quad, 0)
        for u in range(4):
            pltpu.make_async_copy(
                Wb[u], acc_sp.at[didx.at[GRP - 4 + u]], ssc[u]).wait()
        return carry
    lax.fori_loop(0, NGRP, group, 0)
    plsc.subcore_barrier()

    @pl.when(c == 0)
    def _():
        for off, ln in WBCH:
            base = s * ROWS_PER_TILE + off
            pltpu.sync_copy(acc_sp.at[pl.ds(base, ln)], zb.at[pl.ds(0, ln)])
            pltpu.sync_copy(zb.at[pl.ds(0, ln)], accA.at[pl.ds(base, ln)])

    @pl.when(c == 1)
    def _():
        for off, ln in WBCH:
            base = s * ROWS_PER_TILE + off
            pltpu.sync_copy(acc_sp.at[pl.ds(base, ln)], zb.at[pl.ds(0, ln)])
            pltpu.sync_copy(zb.at[pl.ds(0, ln)], accB.at[pl.ds(base, ln)])


def _prep_body(x_ref, o0_ref, o1_ref, i0_ref, i1_ref, xn_ref, sc_ref):
    oc = jnp.maximum(o0_ref[:, :1] + o1_ref[:, :1], 1.0)
    so = lax.rsqrt(oc)
    xn_ref[...] = x_ref[...] * so
    ic = jnp.maximum(i0_ref[:, :1] + i1_ref[:, :1], 1.0)
    sA = lax.rsqrt(ic)
    sc_ref[...] = jnp.concatenate(
        [sA, sA / ic, so, jnp.zeros((BM, 5), jnp.float32)], axis=1)


def _tc_prep(x, o0, o1, i0, i1):
    return pl.pallas_call(
        _prep_body,
        grid=(GRID,),
        in_specs=[
            pl.BlockSpec((BM, D), lambda i: (i, 0)),
            pl.BlockSpec((BM, D), lambda i: (i, 0)),
            pl.BlockSpec((BM, D), lambda i: (i, 0)),
            pl.BlockSpec((BM, D), lambda i: (i, 0)),
            pl.BlockSpec((BM, D), lambda i: (i, 0)),
        ],
        out_specs=(
            pl.BlockSpec((BM, D), lambda i: (i, 0)),
            pl.BlockSpec((BM, 8), lambda i: (i, 0)),
        ),
        out_shape=(
            jax.ShapeDtypeStruct((N, D), jnp.float32),
            jax.ShapeDtypeStruct((N, 8), jnp.float32),
        ),
    )(x, o0, o1, i0, i1)


def _mm_body(x_ref, w_ref, o_ref):
    o_ref[...] = jnp.dot(x_ref[...], w_ref[...],
                         preferred_element_type=jnp.float32)


def _tc_matmul(x, w):
    return pl.pallas_call(
        _mm_body,
        grid=(GRID,),
        in_specs=[
            pl.BlockSpec((BM, D), lambda i: (i, 0)),
            pl.BlockSpec((D, D), lambda i: (0, 0)),
        ],
        out_specs=pl.BlockSpec((BM, D), lambda i: (i, 0)),
        out_shape=jax.ShapeDtypeStruct((N, D), jnp.float32),
    )(x, w)


def _final_body_norm(mm_ref, aA_ref, aB_ref, sc_ref, h_ref, hn_ref):
    h = mm_ref[...] * sc_ref[:, 0:1] + (aA_ref[...] + aB_ref[...]) * sc_ref[:, 1:2]
    h_ref[...] = h
    hn_ref[...] = h * sc_ref[:, 2:3]


def _final_body(mm_ref, aA_ref, aB_ref, sc_ref, h_ref):
    h_ref[...] = (mm_ref[...] * sc_ref[:, 0:1]
                  + (aA_ref[...] + aB_ref[...]) * sc_ref[:, 1:2])


def _tc_final(mm, aA, aB, sc, with_norm):
    body = _final_body_norm if with_norm else _final_body
    out_shape = jax.ShapeDtypeStruct((N, D), jnp.float32)
    out_specs = pl.BlockSpec((BM, D), lambda i: (i, 0))
    if with_norm:
        out_shape = (out_shape, jax.ShapeDtypeStruct((N, D), jnp.float32))
        out_specs = (out_specs, pl.BlockSpec((BM, D), lambda i: (i, 0)))
    return pl.pallas_call(
        body,
        grid=(GRID,),
        in_specs=[
            pl.BlockSpec((BM, D), lambda i: (i, 0)),
            pl.BlockSpec((BM, D), lambda i: (i, 0)),
            pl.BlockSpec((BM, D), lambda i: (i, 0)),
            pl.BlockSpec((BM, 8), lambda i: (i, 0)),
        ],
        out_specs=out_specs,
        out_shape=out_shape,
    )(mm, aA, aB, sc)


def kernel(x, edge_index, etype, weight, loop_w1, loop_w2):
    src2 = edge_index[0].reshape(NW, RW, CH)
    dst2 = edge_index[1].reshape(NW, RW, CH)
    et2 = etype.reshape(NW, RW, CH)
    se2 = jnp.concatenate([src2, et2], axis=2)
    srcC = edge_index[0].reshape(NW, RWC, CCH)
    dstC = edge_index[1].reshape(NW, RWC, CCH)

    o0, i0, o1, i1 = _sc_counts(srcC, dstC)
    xn, sc = _tc_prep(x, o0, o1, i0, i1)

    mm1 = _tc_matmul(x, loop_w1)
    aA, aB = _sc_mp(xn, se2, dst2, weight)
    h, hn = _tc_final(mm1, aA, aB, sc, True)

    mm2 = _tc_matmul(h, loop_w2)
    bA, bB = _sc_mp(hn, se2, dst2, weight)
    return _tc_final(mm2, bA, bB, sc, False)
